# baseline (speedup 1.0000x reference)
import functools

import jax
import jax.numpy as jnp
from jax import lax
from jax.experimental import pallas as pl
from jax.experimental.pallas import tpu as pltpu
from jax.experimental.pallas import tpu_sc as plsc

_NC = 2
_NS = 16
_NW = _NC * _NS
_SUB = 125
_PART = 8


def _pad_rows(n):
    g = 8 * _NS
    return ((n + g - 1) // g) * g


def _mesh():
    return plsc.VectorSubcoreMesh(core_axis_name="c", subcore_axis_name="s")


@functools.lru_cache(maxsize=None)
def _make_count(E, N, W):
    NP = _pad_rows(N)
    CH = E // _NW
    NSUB = CH // _SUB
    ROWS = NP // _NS
    mesh = _mesh()

    @functools.partial(
        pl.kernel, mesh=mesh,
        compiler_params=pltpu.CompilerParams(use_tc_tiling_on_sc=False),
        out_type=jax.ShapeDtypeStruct((_NC, NP, W), jnp.float32),
        scratch_types=[
            pltpu.VMEM((NSUB, _SUB), jnp.int32),
            pltpu.VMEM((_SUB, W), jnp.float32),
            pltpu.VMEM_SHARED((NP, W), jnp.float32),
        ],
    )
    def k(dst_hbm, ones_hbm, zeros_hbm, out_hbm, idx_v, ones_v, acc_sh):
        c = lax.axis_index("c")
        s = lax.axis_index("s")
        wid = s * _NC + c
        row0 = pl.multiple_of(s * ROWS, 8)
        idx0 = pl.multiple_of(wid * NSUB, 8)
        pltpu.sync_copy(zeros_hbm.at[pl.ds(row0, ROWS)],
                        acc_sh.at[pl.ds(row0, ROWS)])
        pltpu.sync_copy(ones_hbm, ones_v)
        pltpu.sync_copy(dst_hbm.at[pl.ds(idx0, NSUB)], idx_v)
        plsc.subcore_barrier()

        def body(j, carry):
            pltpu.sync_copy(ones_v, acc_sh.at[idx_v.at[j]], add=True)
            return carry

        lax.fori_loop(0, NSUB, body, 0)
        plsc.subcore_barrier()
        pltpu.sync_copy(acc_sh.at[pl.ds(row0, ROWS)],
                        out_hbm.at[c, pl.ds(row0, ROWS)])

    return k


@functools.lru_cache(maxsize=None)
def _make_segsum(E, N, D):
    NP = _pad_rows(N)
    CH = E // _NW
    NSUB = CH // _SUB
    NPARTS = NSUB // _PART
    PROWS = _PART * _SUB
    ROWS = NP // _NS
    mesh = _mesh()

    @functools.partial(
        pl.kernel, mesh=mesh,
        compiler_params=pltpu.CompilerParams(use_tc_tiling_on_sc=False),
        out_type=jax.ShapeDtypeStruct((_NC, NP, D), jnp.float32),
        scratch_types=[
            pltpu.VMEM((NSUB, _SUB), jnp.int32),
            pltpu.VMEM((NSUB, _SUB), jnp.int32),
            pltpu.VMEM((PROWS, D), jnp.float32),
            pltpu.SemaphoreType.DMA,
            pltpu.VMEM_SHARED((NP, D), jnp.float32),
        ],
    )
    def k(table_hbm, src_hbm, dst_hbm, zeros_hbm, out_hbm,
          src_v, dst_v, rows_v, sem, acc_sh):
        c = lax.axis_index("c")
        s = lax.axis_index("s")
        wid = s * _NC + c
        row0 = pl.multiple_of(s * ROWS, 8)
        idx0 = pl.multiple_of(wid * NSUB, 8)
        pltpu.sync_copy(zeros_hbm.at[pl.ds(row0, ROWS)],
                        acc_sh.at[pl.ds(row0, ROWS)])
        pltpu.sync_copy(src_hbm.at[pl.ds(idx0, NSUB)], src_v)
        pltpu.sync_copy(dst_hbm.at[pl.ds(idx0, NSUB)], dst_v)
        plsc.subcore_barrier()

        for part in range(NPARTS):
            base = part * _PART

            def fire(j, carry):
                pltpu.async_copy(table_hbm.at[src_v.at[base + j]],
                                 rows_v.at[pl.ds(j * _SUB, _SUB)], sem)
                return carry

            lax.fori_loop(0, _PART, fire, 0)
            pltpu.make_async_copy(table_hbm.at[pl.ds(0, PROWS)],
                                  rows_v, sem).wait()

            def scat(j, carry):
                pltpu.sync_copy(rows_v.at[pl.ds(j * _SUB, _SUB)],
                                acc_sh.at[dst_v.at[base + j]], add=True)
                return carry

            lax.fori_loop(0, _PART, scat, 0)

        plsc.subcore_barrier()
        pltpu.sync_copy(acc_sh.at[pl.ds(row0, ROWS)],
                        out_hbm.at[c, pl.ds(row0, ROWS)])

    return k


@functools.lru_cache(maxsize=None)
def _make_gather(E, N, D):
    CH = E // _NW
    NSUB = CH // _SUB
    NPARTS = NSUB // _PART
    PROWS = _PART * _SUB
    mesh = _mesh()

    @functools.partial(
        pl.kernel, mesh=mesh,
        compiler_params=pltpu.CompilerParams(use_tc_tiling_on_sc=False),
        out_type=jax.ShapeDtypeStruct((E, D), jnp.float32),
        scratch_types=[
            pltpu.VMEM((NSUB, _SUB), jnp.int32),
            pltpu.VMEM((PROWS, D), jnp.float32),
            pltpu.SemaphoreType.DMA,
        ],
    )
    def k(table_hbm, src_hbm, out_hbm, src_v, rows_v, sem):
        c = lax.axis_index("c")
        s = lax.axis_index("s")
        wid = s * _NC + c
        idx0 = pl.multiple_of(wid * NSUB, 8)
        pltpu.sync_copy(src_hbm.at[pl.ds(idx0, NSUB)], src_v)
        for part in range(NPARTS):
            base = part * _PART

            def fire(j, carry):
                pltpu.async_copy(table_hbm.at[src_v.at[base + j]],
                                 rows_v.at[pl.ds(j * _SUB, _SUB)], sem)
                return carry

            lax.fori_loop(0, _PART, fire, 0)
            pltpu.make_async_copy(table_hbm.at[pl.ds(0, PROWS)],
                                  rows_v, sem).wait()
            out0 = pl.multiple_of(wid * CH + part * PROWS, 8)
            pltpu.sync_copy(rows_v, out_hbm.at[pl.ds(out0, PROWS)])

    return k


@functools.lru_cache(maxsize=None)
def _make_scatter_rows(E, N, D):
    NP = _pad_rows(N)
    CH = E // _NW
    NSUB = CH // _SUB
    NPARTS = NSUB // _PART
    PROWS = _PART * _SUB
    ROWS = NP // _NS
    mesh = _mesh()

    @functools.partial(
        pl.kernel, mesh=mesh,
        compiler_params=pltpu.CompilerParams(use_tc_tiling_on_sc=False),
        out_type=jax.ShapeDtypeStruct((_NC, NP, D), jnp.float32),
        scratch_types=[
            pltpu.VMEM((NSUB, _SUB), jnp.int32),
            pltpu.VMEM((PROWS, D), jnp.float32),
            pltpu.VMEM_SHARED((NP, D), jnp.float32),
        ],
    )
    def k(rows_hbm, dst_hbm, zeros_hbm, out_hbm, dst_v, rows_v, acc_sh):
        c = lax.axis_index("c")
        s = lax.axis_index("s")
        wid = s * _NC + c
        row0 = pl.multiple_of(s * ROWS, 8)
        idx0 = pl.multiple_of(wid * NSUB, 8)
        pltpu.sync_copy(zeros_hbm.at[pl.ds(row0, ROWS)],
                        acc_sh.at[pl.ds(row0, ROWS)])
        pltpu.sync_copy(dst_hbm.at[pl.ds(idx0, NSUB)], dst_v)
        plsc.subcore_barrier()

        for part in range(NPARTS):
            base = part * _PART
            in0 = pl.multiple_of(wid * CH + part * PROWS, 8)
            pltpu.sync_copy(rows_hbm.at[pl.ds(in0, PROWS)], rows_v)

            def scat(j, carry):
                pltpu.sync_copy(rows_v.at[pl.ds(j * _SUB, _SUB)],
                                acc_sh.at[dst_v.at[base + j]], add=True)
                return carry

            lax.fori_loop(0, _PART, scat, 0)

        plsc.subcore_barrier()
        pltpu.sync_copy(acc_sh.at[pl.ds(row0, ROWS)],
                        out_hbm.at[c, pl.ds(row0, ROWS)])

    return k


def _mm(x, w, b, act=None, blk=1000):
    M, K = x.shape
    Nw = w.shape[1]

    def body(x_ref, w_ref, b_ref, o_ref):
        acc = jnp.dot(x_ref[...], w_ref[...],
                      preferred_element_type=jnp.float32) + b_ref[...]
        if act == "relu":
            acc = jnp.maximum(acc, 0.0)
        elif act == "tanh":
            acc = jnp.tanh(acc)
        o_ref[...] = acc

    return pl.pallas_call(
        body,
        grid=(M // blk,),
        in_specs=[
            pl.BlockSpec((blk, K), lambda i: (i, 0)),
            pl.BlockSpec((K, Nw), lambda i: (0, 0)),
            pl.BlockSpec((1, Nw), lambda i: (0, 0)),
        ],
        out_specs=pl.BlockSpec((blk, Nw), lambda i: (i, 0)),
        out_shape=jax.ShapeDtypeStruct((M, Nw), jnp.float32),
    )(x, w, b)


def _gcn_pre(x, w, cntp, blk=1000):
    M, K = x.shape
    Nw = w.shape[1]
    Wc = cntp.shape[2]

    def body(x_ref, w_ref, c_ref, o_ref):
        cnt = c_ref[0, :, 0:1] + c_ref[1, :, 0:1]
        dinv = lax.rsqrt(1.0 + cnt)
        o_ref[...] = dinv * jnp.dot(x_ref[...], w_ref[...],
                                    preferred_element_type=jnp.float32)

    return pl.pallas_call(
        body,
        grid=(M // blk,),
        in_specs=[
            pl.BlockSpec((blk, K), lambda i: (i, 0)),
            pl.BlockSpec((K, Nw), lambda i: (0, 0)),
            pl.BlockSpec((2, blk, Wc), lambda i: (0, i, 0)),
        ],
        out_specs=pl.BlockSpec((blk, Nw), lambda i: (i, 0)),
        out_shape=jax.ShapeDtypeStruct((M, Nw), jnp.float32),
    )(x, w, cntp)


def _gcn_step(sp, a, cntp, b, w=None, act=None, blk=1000):
    M, D = a.shape
    Wc = cntp.shape[2]
    has_w = w is not None
    Nw = w.shape[1] if has_w else D

    def body(*refs):
        if has_w:
            s_ref, a_ref, c_ref, b_ref, w_ref, o_ref = refs
        else:
            s_ref, a_ref, c_ref, b_ref, o_ref = refs
        cnt = c_ref[0, :, 0:1] + c_ref[1, :, 0:1]
        dinv = lax.rsqrt(1.0 + cnt)
        z = dinv * (s_ref[0] + s_ref[1] + a_ref[...]) + b_ref[...]
        if act == "relu":
            z = jnp.maximum(z, 0.0)
        if has_w:
            z = dinv * jnp.dot(z, w_ref[...],
                               preferred_element_type=jnp.float32)
        o_ref[...] = z

    in_specs = [
        pl.BlockSpec((2, blk, D), lambda i: (0, i, 0)),
        pl.BlockSpec((blk, D), lambda i: (i, 0)),
        pl.BlockSpec((2, blk, Wc), lambda i: (0, i, 0)),
        pl.BlockSpec((1, D), lambda i: (0, 0)),
    ]
    args = [sp, a, cntp, b]
    if has_w:
        in_specs.append(pl.BlockSpec((D, Nw), lambda i: (0, 0)))
        args.append(w)

    return pl.pallas_call(
        body,
        grid=(M // blk,),
        in_specs=in_specs,
        out_specs=pl.BlockSpec((blk, Nw), lambda i: (i, 0)),
        out_shape=jax.ShapeDtypeStruct((M, Nw), jnp.float32),
    )(*args)


def _nnconv_msg(ea, zs, wn1, bn1, wstack, blk=1000):
    E = ea.shape[0]
    K1 = wn1.shape[0]
    K2 = wn1.shape[1]
    D = zs.shape[1]

    def body(ea_ref, zs_ref, w1_ref, b1_ref, *rest):
        ws_refs = rest[:K2 + 1]
        o_ref = rest[K2 + 1]
        a = jnp.maximum(jnp.dot(ea_ref[...], w1_ref[...],
                                preferred_element_type=jnp.float32)
                        + b1_ref[...], 0.0)
        zsb = zs_ref[...]
        m = jnp.dot(zsb, ws_refs[K2][...], preferred_element_type=jnp.float32)
        for k in range(K2):
            m = m + a[:, k:k + 1] * jnp.dot(zsb, ws_refs[k][...],
                                            preferred_element_type=jnp.float32)
        o_ref[...] = m

    return pl.pallas_call(
        body,
        grid=(E // blk,),
        in_specs=[
            pl.BlockSpec((blk, K1), lambda i: (i, 0)),
            pl.BlockSpec((blk, D), lambda i: (i, 0)),
            pl.BlockSpec((K1, K2), lambda i: (0, 0)),
            pl.BlockSpec((1, K2), lambda i: (0, 0)),
        ] + [pl.BlockSpec((D, D), lambda i: (0, 0)) for _ in range(K2 + 1)],
        out_specs=pl.BlockSpec((blk, D), lambda i: (i, 0)),
        out_shape=jax.ShapeDtypeStruct((E, D), jnp.float32),
    )(ea, zs, wn1, bn1, *[wstack[k] for k in range(K2 + 1)])


def _nnconv_combine(mp, cntp, z, wroot, broot, blk=1000):
    M, D = z.shape
    Wc = cntp.shape[2]

    def body(m_ref, c_ref, z_ref, w_ref, b_ref, o_ref):
        cnt = c_ref[0, :, 0:1] + c_ref[1, :, 0:1]
        inv = 1.0 / jnp.maximum(cnt, 1.0)
        aggr = (m_ref[0] + m_ref[1]) * inv
        o_ref[...] = jnp.tanh(aggr + jnp.dot(z_ref[...], w_ref[...],
                                             preferred_element_type=jnp.float32)
                              + b_ref[...])

    return pl.pallas_call(
        body,
        grid=(M // blk,),
        in_specs=[
            pl.BlockSpec((2, blk, D), lambda i: (0, i, 0)),
            pl.BlockSpec((2, blk, Wc), lambda i: (0, i, 0)),
            pl.BlockSpec((blk, D), lambda i: (i, 0)),
            pl.BlockSpec((D, D), lambda i: (0, 0)),
            pl.BlockSpec((1, D), lambda i: (0, 0)),
        ],
        out_specs=pl.BlockSpec((blk, D), lambda i: (i, 0)),
        out_shape=jax.ShapeDtypeStruct((M, D), jnp.float32),
    )(mp, cntp, z, wroot, broot)


def _gru(gr4, gz4, gn4, wr, wz, wn, br, bz, bn, T, H):
    M = gr4.shape[0]

    def sig(x):
        return 0.5 + 0.5 * jnp.tanh(0.5 * x)

    def body(gr_ref, gz_ref, gn_ref, wr_ref, wz_ref, wn_ref,
             br_ref, bz_ref, bn_ref, o_ref):
        wrv = wr_ref[...]
        wzv = wz_ref[...]
        wnv = wn_ref[...]
        brv = br_ref[...]
        bzv = bz_ref[...]
        bnv = bn_ref[...]

        def outer(m, h):
            tr = gr_ref[m]
            tz = gz_ref[m]
            tn = gn_ref[m]
            outs = []
            for r in range(4):
                sl = slice(2 * r, 2 * r + 2)
                hb = h.astype(jnp.bfloat16)
                hr = jnp.dot(hb, wrv, preferred_element_type=jnp.float32) + brv
                hz = jnp.dot(hb, wzv, preferred_element_type=jnp.float32) + bzv
                hn = jnp.dot(hb, wnv, preferred_element_type=jnp.float32) + bnv
                rr = sig(tr[sl] + hr)
                zg = sig(tz[sl] + hz)
                nn = jnp.tanh(tn[sl] + rr * hn)
                h = (1.0 - zg) * nn + zg * h
                outs.append(h)
            o_ref[m] = jnp.concatenate(outs, axis=0)
            return h

        lax.fori_loop(0, M, outer, jnp.zeros((T, H), jnp.float32))

    return pl.pallas_call(
        body,
        out_shape=jax.ShapeDtypeStruct((M, 8, H), jnp.float32),
    )(gr4, gz4, gn4, wr, wz, wn, br, bz, bn)


def kernel(x, eis, eas, W1, b1, W2, b2, W3, b3, Wn1, bn1, Wn2, bn2,
           Wroot, broot, Wih, Whh, bih, bhh, Wl, bl):
    N, IN_DIM = x.shape
    T, _, E = eis.shape
    H = W1.shape[1]
    K2 = Wn1.shape[1]
    CW = 16

    count_k = _make_count(E, N, CW)
    segsum_k = _make_segsum(E, N, H)
    gather_k = _make_gather(E, N, H)
    scatter_k = _make_scatter_rows(E, N, H)

    NP = _pad_rows(N)
    ones_sub = jnp.ones((_SUB, CW), jnp.float32)
    zeros_cnt = jnp.zeros((NP, CW), jnp.float32)
    zeros_h = jnp.zeros((NP, H), jnp.float32)

    wstack = jnp.concatenate(
        [Wn2.reshape(K2, H, H), bn2.reshape(1, H, H)], axis=0)

    b1r = b1.reshape(1, H)
    b2r = b2.reshape(1, H)
    b3r = b3.reshape(1, H)
    bn1r = bn1.reshape(1, K2)
    brootr = broot.reshape(1, H)

    zs_list = []
    for t in range(T):
        src = eis[t, 0].reshape(E // _SUB, _SUB)
        dst = eis[t, 1].reshape(E // _SUB, _SUB)

        cntp = count_k(dst, ones_sub, zeros_cnt)

        a1 = _gcn_pre(x, W1, cntp)
        s1 = segsum_k(a1, src, dst, zeros_h)
        a2 = _gcn_step(s1, a1, cntp, b1r, w=W2, act=None)
        s2 = segsum_k(a2, src, dst, zeros_h)
        a3 = _gcn_step(s2, a2, cntp, b2r, w=W3, act="relu")
        s3 = segsum_k(a3, src, dst, zeros_h)
        z3 = _gcn_step(s3, a3, cntp, b3r, w=None, act="relu")

        zsrc = gather_k(z3, src)
        msg = _nnconv_msg(eas[t], zsrc, Wn1, bn1r, wstack)
        mp = scatter_k(msg, dst, zeros_h)
        zt = _nnconv_combine(mp, cntp, z3, Wroot, brootr)
        zs_list.append(zt)

    zseq = jnp.stack(zs_list, axis=1)
    zf = zseq.reshape(N * T, H)
    M4 = N * T // 8
    wihT = Wih.T
    gr = _mm(zf, wihT[:, :H], bih[:H].reshape(1, H)).reshape(M4, 8, H)
    gz = _mm(zf, wihT[:, H:2 * H], bih[H:2 * H].reshape(1, H)).reshape(M4, 8, H)
    gn = _mm(zf, wihT[:, 2 * H:], bih[2 * H:].reshape(1, H)).reshape(M4, 8, H)
    whhT = Whh.T
    whhT16 = whhT.astype(jnp.bfloat16)
    hs = _gru(gr, gz, gn,
              whhT16[:, :H], whhT16[:, H:2 * H], whhT16[:, 2 * H:],
              bhh[:H].reshape(1, H), bhh[H:2 * H].reshape(1, H),
              bhh[2 * H:].reshape(1, H), T, H)
    out = _mm(hs.reshape(N * T, H), Wl, bl.reshape(1, Wl.shape[1]))
    return out.reshape(N, T, Wl.shape[1])


# speedup vs baseline: 1.0157x; 1.0157x over previous
import functools

import jax
import jax.numpy as jnp
from jax import lax
from jax.experimental import pallas as pl
from jax.experimental.pallas import tpu as pltpu
from jax.experimental.pallas import tpu_sc as plsc

_NC = 2
_NS = 16
_NW = _NC * _NS
_SUB = 125
_PART = 8


def _pad_rows(n):
    g = 8 * _NS
    return ((n + g - 1) // g) * g


def _mesh():
    return plsc.VectorSubcoreMesh(core_axis_name="c", subcore_axis_name="s")


@functools.lru_cache(maxsize=None)
def _make_count(E, N, W):
    NP = _pad_rows(N)
    CH = E // _NW
    NSUB = CH // _SUB
    ROWS = NP // _NS
    mesh = _mesh()

    @functools.partial(
        pl.kernel, mesh=mesh,
        compiler_params=pltpu.CompilerParams(use_tc_tiling_on_sc=False),
        out_type=jax.ShapeDtypeStruct((_NC, NP, W), jnp.float32),
        scratch_types=[
            pltpu.VMEM((NSUB, _SUB), jnp.int32),
            pltpu.VMEM((_SUB, W), jnp.float32),
            pltpu.VMEM_SHARED((NP, W), jnp.float32),
        ],
    )
    def k(dst_hbm, ones_hbm, zeros_hbm, out_hbm, idx_v, ones_v, acc_sh):
        c = lax.axis_index("c")
        s = lax.axis_index("s")
        wid = s * _NC + c
        row0 = pl.multiple_of(s * ROWS, 8)
        idx0 = pl.multiple_of(wid * NSUB, 8)
        pltpu.sync_copy(zeros_hbm.at[pl.ds(row0, ROWS)],
                        acc_sh.at[pl.ds(row0, ROWS)])
        pltpu.sync_copy(ones_hbm, ones_v)
        pltpu.sync_copy(dst_hbm.at[pl.ds(idx0, NSUB)], idx_v)
        plsc.subcore_barrier()

        def body(j, carry):
            pltpu.sync_copy(ones_v, acc_sh.at[idx_v.at[j]], add=True)
            return carry

        lax.fori_loop(0, NSUB, body, 0)
        plsc.subcore_barrier()
        pltpu.sync_copy(acc_sh.at[pl.ds(row0, ROWS)],
                        out_hbm.at[c, pl.ds(row0, ROWS)])

    return k


@functools.lru_cache(maxsize=None)
def _make_segsum(E, N, D):
    NP = _pad_rows(N)
    CH = E // _NW
    NSUB = CH // _SUB
    NPARTS = NSUB // _PART
    PROWS = _PART * _SUB
    ROWS = NP // _NS
    mesh = _mesh()

    @functools.partial(
        pl.kernel, mesh=mesh,
        compiler_params=pltpu.CompilerParams(use_tc_tiling_on_sc=False),
        out_type=jax.ShapeDtypeStruct((_NC, NP, D), jnp.float32),
        scratch_types=[
            pltpu.VMEM((NSUB, _SUB), jnp.int32),
            pltpu.VMEM((NSUB, _SUB), jnp.int32),
            pltpu.VMEM((PROWS, D), jnp.float32),
            pltpu.SemaphoreType.DMA,
            pltpu.VMEM_SHARED((NP, D), jnp.float32),
        ],
    )
    def k(table_hbm, src_hbm, dst_hbm, zeros_hbm, out_hbm,
          src_v, dst_v, rows_v, sem, acc_sh):
        c = lax.axis_index("c")
        s = lax.axis_index("s")
        wid = s * _NC + c
        row0 = pl.multiple_of(s * ROWS, 8)
        idx0 = pl.multiple_of(wid * NSUB, 8)
        pltpu.sync_copy(zeros_hbm.at[pl.ds(row0, ROWS)],
                        acc_sh.at[pl.ds(row0, ROWS)])
        pltpu.sync_copy(src_hbm.at[pl.ds(idx0, NSUB)], src_v)
        pltpu.sync_copy(dst_hbm.at[pl.ds(idx0, NSUB)], dst_v)
        plsc.subcore_barrier()

        for part in range(NPARTS):
            base = part * _PART

            def fire(j, carry):
                pltpu.async_copy(table_hbm.at[src_v.at[base + j]],
                                 rows_v.at[pl.ds(j * _SUB, _SUB)], sem)
                return carry

            lax.fori_loop(0, _PART, fire, 0)
            pltpu.make_async_copy(table_hbm.at[pl.ds(0, PROWS)],
                                  rows_v, sem).wait()

            def scat(j, carry):
                pltpu.sync_copy(rows_v.at[pl.ds(j * _SUB, _SUB)],
                                acc_sh.at[dst_v.at[base + j]], add=True)
                return carry

            lax.fori_loop(0, _PART, scat, 0)

        plsc.subcore_barrier()
        pltpu.sync_copy(acc_sh.at[pl.ds(row0, ROWS)],
                        out_hbm.at[c, pl.ds(row0, ROWS)])

    return k


@functools.lru_cache(maxsize=None)
def _make_gather(E, N, D):
    CH = E // _NW
    NSUB = CH // _SUB
    NPARTS = NSUB // _PART
    PROWS = _PART * _SUB
    mesh = _mesh()

    @functools.partial(
        pl.kernel, mesh=mesh,
        compiler_params=pltpu.CompilerParams(use_tc_tiling_on_sc=False),
        out_type=jax.ShapeDtypeStruct((E, D), jnp.float32),
        scratch_types=[
            pltpu.VMEM((NSUB, _SUB), jnp.int32),
            pltpu.VMEM((PROWS, D), jnp.float32),
            pltpu.SemaphoreType.DMA,
        ],
    )
    def k(table_hbm, src_hbm, out_hbm, src_v, rows_v, sem):
        c = lax.axis_index("c")
        s = lax.axis_index("s")
        wid = s * _NC + c
        idx0 = pl.multiple_of(wid * NSUB, 8)
        pltpu.sync_copy(src_hbm.at[pl.ds(idx0, NSUB)], src_v)
        for part in range(NPARTS):
            base = part * _PART

            def fire(j, carry):
                pltpu.async_copy(table_hbm.at[src_v.at[base + j]],
                                 rows_v.at[pl.ds(j * _SUB, _SUB)], sem)
                return carry

            lax.fori_loop(0, _PART, fire, 0)
            pltpu.make_async_copy(table_hbm.at[pl.ds(0, PROWS)],
                                  rows_v, sem).wait()
            out0 = pl.multiple_of(wid * CH + part * PROWS, 8)
            pltpu.sync_copy(rows_v, out_hbm.at[pl.ds(out0, PROWS)])

    return k


@functools.lru_cache(maxsize=None)
def _make_scatter_rows(E, N, D):
    NP = _pad_rows(N)
    CH = E // _NW
    NSUB = CH // _SUB
    NPARTS = NSUB // _PART
    PROWS = _PART * _SUB
    ROWS = NP // _NS
    mesh = _mesh()

    @functools.partial(
        pl.kernel, mesh=mesh,
        compiler_params=pltpu.CompilerParams(use_tc_tiling_on_sc=False),
        out_type=jax.ShapeDtypeStruct((_NC, NP, D), jnp.float32),
        scratch_types=[
            pltpu.VMEM((NSUB, _SUB), jnp.int32),
            pltpu.VMEM((PROWS, D), jnp.float32),
            pltpu.VMEM_SHARED((NP, D), jnp.float32),
        ],
    )
    def k(rows_hbm, dst_hbm, zeros_hbm, out_hbm, dst_v, rows_v, acc_sh):
        c = lax.axis_index("c")
        s = lax.axis_index("s")
        wid = s * _NC + c
        row0 = pl.multiple_of(s * ROWS, 8)
        idx0 = pl.multiple_of(wid * NSUB, 8)
        pltpu.sync_copy(zeros_hbm.at[pl.ds(row0, ROWS)],
                        acc_sh.at[pl.ds(row0, ROWS)])
        pltpu.sync_copy(dst_hbm.at[pl.ds(idx0, NSUB)], dst_v)
        plsc.subcore_barrier()

        for part in range(NPARTS):
            base = part * _PART
            in0 = pl.multiple_of(wid * CH + part * PROWS, 8)
            pltpu.sync_copy(rows_hbm.at[pl.ds(in0, PROWS)], rows_v)

            def scat(j, carry):
                pltpu.sync_copy(rows_v.at[pl.ds(j * _SUB, _SUB)],
                                acc_sh.at[dst_v.at[base + j]], add=True)
                return carry

            lax.fori_loop(0, _PART, scat, 0)

        plsc.subcore_barrier()
        pltpu.sync_copy(acc_sh.at[pl.ds(row0, ROWS)],
                        out_hbm.at[c, pl.ds(row0, ROWS)])

    return k


def _mm(x, w, b, act=None, blk=1000):
    M, K = x.shape
    Nw = w.shape[1]

    def body(x_ref, w_ref, b_ref, o_ref):
        acc = jnp.dot(x_ref[...], w_ref[...],
                      preferred_element_type=jnp.float32) + b_ref[...]
        if act == "relu":
            acc = jnp.maximum(acc, 0.0)
        elif act == "tanh":
            acc = jnp.tanh(acc)
        o_ref[...] = acc

    return pl.pallas_call(
        body,
        grid=(M // blk,),
        in_specs=[
            pl.BlockSpec((blk, K), lambda i: (i, 0)),
            pl.BlockSpec((K, Nw), lambda i: (0, 0)),
            pl.BlockSpec((1, Nw), lambda i: (0, 0)),
        ],
        out_specs=pl.BlockSpec((blk, Nw), lambda i: (i, 0)),
        out_shape=jax.ShapeDtypeStruct((M, Nw), jnp.float32),
    )(x, w, b)


def _gcn_pre(x, w, cntp, blk=1000):
    M, K = x.shape
    Nw = w.shape[1]
    Wc = cntp.shape[2]

    def body(x_ref, w_ref, c_ref, o_ref):
        cnt = c_ref[0, :, 0:1] + c_ref[1, :, 0:1]
        dinv = lax.rsqrt(1.0 + cnt)
        o_ref[...] = dinv * jnp.dot(x_ref[...], w_ref[...],
                                    preferred_element_type=jnp.float32)

    return pl.pallas_call(
        body,
        grid=(M // blk,),
        in_specs=[
            pl.BlockSpec((blk, K), lambda i: (i, 0)),
            pl.BlockSpec((K, Nw), lambda i: (0, 0)),
            pl.BlockSpec((2, blk, Wc), lambda i: (0, i, 0)),
        ],
        out_specs=pl.BlockSpec((blk, Nw), lambda i: (i, 0)),
        out_shape=jax.ShapeDtypeStruct((M, Nw), jnp.float32),
    )(x, w, cntp)


def _gcn_step(sp, a, cntp, b, w=None, act=None, blk=1000):
    M, D = a.shape
    Wc = cntp.shape[2]
    has_w = w is not None
    Nw = w.shape[1] if has_w else D

    def body(*refs):
        if has_w:
            s_ref, a_ref, c_ref, b_ref, w_ref, o_ref = refs
        else:
            s_ref, a_ref, c_ref, b_ref, o_ref = refs
        cnt = c_ref[0, :, 0:1] + c_ref[1, :, 0:1]
        dinv = lax.rsqrt(1.0 + cnt)
        z = dinv * (s_ref[0] + s_ref[1] + a_ref[...]) + b_ref[...]
        if act == "relu":
            z = jnp.maximum(z, 0.0)
        if has_w:
            z = dinv * jnp.dot(z, w_ref[...],
                               preferred_element_type=jnp.float32)
        o_ref[...] = z

    in_specs = [
        pl.BlockSpec((2, blk, D), lambda i: (0, i, 0)),
        pl.BlockSpec((blk, D), lambda i: (i, 0)),
        pl.BlockSpec((2, blk, Wc), lambda i: (0, i, 0)),
        pl.BlockSpec((1, D), lambda i: (0, 0)),
    ]
    args = [sp, a, cntp, b]
    if has_w:
        in_specs.append(pl.BlockSpec((D, Nw), lambda i: (0, 0)))
        args.append(w)

    return pl.pallas_call(
        body,
        grid=(M // blk,),
        in_specs=in_specs,
        out_specs=pl.BlockSpec((blk, Nw), lambda i: (i, 0)),
        out_shape=jax.ShapeDtypeStruct((M, Nw), jnp.float32),
    )(*args)


def _nnconv_msg(ea, zs, wn1, bn1, wstack, blk=1000):
    E = ea.shape[0]
    K1 = wn1.shape[0]
    K2 = wn1.shape[1]
    D = zs.shape[1]

    def body(ea_ref, zs_ref, w1_ref, b1_ref, *rest):
        ws_refs = rest[:K2 + 1]
        o_ref = rest[K2 + 1]
        a = jnp.maximum(jnp.dot(ea_ref[...], w1_ref[...],
                                preferred_element_type=jnp.float32)
                        + b1_ref[...], 0.0)
        zsb = zs_ref[...]
        m = jnp.dot(zsb, ws_refs[K2][...], preferred_element_type=jnp.float32)
        for k in range(K2):
            m = m + a[:, k:k + 1] * jnp.dot(zsb, ws_refs[k][...],
                                            preferred_element_type=jnp.float32)
        o_ref[...] = m

    return pl.pallas_call(
        body,
        grid=(E // blk,),
        in_specs=[
            pl.BlockSpec((blk, K1), lambda i: (i, 0)),
            pl.BlockSpec((blk, D), lambda i: (i, 0)),
            pl.BlockSpec((K1, K2), lambda i: (0, 0)),
            pl.BlockSpec((1, K2), lambda i: (0, 0)),
        ] + [pl.BlockSpec((D, D), lambda i: (0, 0)) for _ in range(K2 + 1)],
        out_specs=pl.BlockSpec((blk, D), lambda i: (i, 0)),
        out_shape=jax.ShapeDtypeStruct((E, D), jnp.float32),
    )(ea, zs, wn1, bn1, *[wstack[k] for k in range(K2 + 1)])


def _nnconv_combine(mp, cntp, z, wroot, broot, blk=1000):
    M, D = z.shape
    Wc = cntp.shape[2]

    def body(m_ref, c_ref, z_ref, w_ref, b_ref, o_ref):
        cnt = c_ref[0, :, 0:1] + c_ref[1, :, 0:1]
        inv = 1.0 / jnp.maximum(cnt, 1.0)
        aggr = (m_ref[0] + m_ref[1]) * inv
        o_ref[...] = jnp.tanh(aggr + jnp.dot(z_ref[...], w_ref[...],
                                             preferred_element_type=jnp.float32)
                              + b_ref[...])

    return pl.pallas_call(
        body,
        grid=(M // blk,),
        in_specs=[
            pl.BlockSpec((2, blk, D), lambda i: (0, i, 0)),
            pl.BlockSpec((2, blk, Wc), lambda i: (0, i, 0)),
            pl.BlockSpec((blk, D), lambda i: (i, 0)),
            pl.BlockSpec((D, D), lambda i: (0, 0)),
            pl.BlockSpec((1, D), lambda i: (0, 0)),
        ],
        out_specs=pl.BlockSpec((blk, D), lambda i: (i, 0)),
        out_shape=jax.ShapeDtypeStruct((M, D), jnp.float32),
    )(mp, cntp, z, wroot, broot)


def _gru(gr4, gz4, gn4, wr, wz, wn, bn, T, H):
    M = gr4.shape[0]

    def body(gr_ref, gz_ref, gn_ref, wr_ref, wz_ref, wn_ref, bn_ref, o_ref):
        wrv = wr_ref[...]
        wzv = wz_ref[...]
        wnv = wn_ref[...]
        bnv = bn_ref[...]

        def outer(m, h):
            tr = gr_ref[m]
            tz = gz_ref[m]
            tn = gn_ref[m]
            outs = []
            for r in range(4):
                sl = slice(2 * r, 2 * r + 2)
                hr = jnp.dot(h, wrv, preferred_element_type=jnp.float32)
                hz = jnp.dot(h, wzv, preferred_element_type=jnp.float32)
                hn = jnp.dot(h, wnv, preferred_element_type=jnp.float32) + bnv
                rr = 0.5 + 0.5 * jnp.tanh(tr[sl] + hr)
                zg = 0.5 + 0.5 * jnp.tanh(tz[sl] + hz)
                nn = jnp.tanh(tn[sl] + rr * hn)
                h = (1.0 - zg) * nn + zg * h
                outs.append(h)
            o_ref[m] = jnp.concatenate(outs, axis=0)
            return h

        lax.fori_loop(0, M, outer, jnp.zeros((T, H), jnp.float32))

    return pl.pallas_call(
        body,
        out_shape=jax.ShapeDtypeStruct((M, 8, H), jnp.float32),
    )(gr4, gz4, gn4, wr, wz, wn, bn)


def kernel(x, eis, eas, W1, b1, W2, b2, W3, b3, Wn1, bn1, Wn2, bn2,
           Wroot, broot, Wih, Whh, bih, bhh, Wl, bl):
    N, IN_DIM = x.shape
    T, _, E = eis.shape
    H = W1.shape[1]
    K2 = Wn1.shape[1]
    CW = 16

    count_k = _make_count(E, N, CW)
    segsum_k = _make_segsum(E, N, H)
    gather_k = _make_gather(E, N, H)
    scatter_k = _make_scatter_rows(E, N, H)

    NP = _pad_rows(N)
    ones_sub = jnp.ones((_SUB, CW), jnp.float32)
    zeros_cnt = jnp.zeros((NP, CW), jnp.float32)
    zeros_h = jnp.zeros((NP, H), jnp.float32)

    wstack = jnp.concatenate(
        [Wn2.reshape(K2, H, H), bn2.reshape(1, H, H)], axis=0)

    b1r = b1.reshape(1, H)
    b2r = b2.reshape(1, H)
    b3r = b3.reshape(1, H)
    bn1r = bn1.reshape(1, K2)
    brootr = broot.reshape(1, H)

    zs_list = []
    for t in range(T):
        src = eis[t, 0].reshape(E // _SUB, _SUB)
        dst = eis[t, 1].reshape(E // _SUB, _SUB)

        cntp = count_k(dst, ones_sub, zeros_cnt)

        a1 = _gcn_pre(x, W1, cntp)
        s1 = segsum_k(a1, src, dst, zeros_h)
        a2 = _gcn_step(s1, a1, cntp, b1r, w=W2, act=None)
        s2 = segsum_k(a2, src, dst, zeros_h)
        a3 = _gcn_step(s2, a2, cntp, b2r, w=W3, act="relu")
        s3 = segsum_k(a3, src, dst, zeros_h)
        z3 = _gcn_step(s3, a3, cntp, b3r, w=None, act="relu")

        zsrc = gather_k(z3, src)
        msg = _nnconv_msg(eas[t], zsrc, Wn1, bn1r, wstack)
        mp = scatter_k(msg, dst, zeros_h)
        zt = _nnconv_combine(mp, cntp, z3, Wroot, brootr)
        zs_list.append(zt)

    zseq = jnp.stack(zs_list, axis=1)
    zf = zseq.reshape(N * T, H)
    M4 = N * T // 8
    wihT = Wih.T
    whhT = Whh.T
    gr = _mm(zf, 0.5 * wihT[:, :H],
             (0.5 * (bih[:H] + bhh[:H])).reshape(1, H)).reshape(M4, 8, H)
    gz = _mm(zf, 0.5 * wihT[:, H:2 * H],
             (0.5 * (bih[H:2 * H] + bhh[H:2 * H])).reshape(1, H)).reshape(M4, 8, H)
    gn = _mm(zf, wihT[:, 2 * H:], bih[2 * H:].reshape(1, H)).reshape(M4, 8, H)
    hs = _gru(gr, gz, gn,
              0.5 * whhT[:, :H], 0.5 * whhT[:, H:2 * H], whhT[:, 2 * H:],
              bhh[2 * H:].reshape(1, H), T, H)
    out = _mm(hs.reshape(N * T, H), Wl, bl.reshape(1, Wl.shape[1]))
    return out.reshape(N, T, Wl.shape[1])
